# baseline (device time: 35851 ns/iter reference)
import jax
import jax.numpy as jnp
from jax import lax
from jax.experimental import pallas as pl
from jax.experimental.pallas import tpu as pltpu

N_DEV = 4
B = 2
SQ = 256
SKV = 256
HQ = 4
DH = 64
D_MODEL = 512
D_QK = HQ * DH
WINDOW = 128
NGLOBAL = 32
SCALE = 0.125
NEG = -1e9


def kernel(x, Wq, K_ext, V_ext, Wo):
    k2 = K_ext.reshape(B, SKV, D_QK)
    v2 = V_ext.reshape(B, SKV, D_QK)

    def body(x_ref, wq_ref, k_ref, v_ref, wo_ref, out_ref,
             kg_ref, vg_ref, ksend, krecv, vsend, vrecv):
        my = lax.axis_index("i")
        left = lax.rem(my + N_DEV - 1, N_DEV)
        right = lax.rem(my + 1, N_DEV)

        kg_ref[0] = k_ref[...].astype(jnp.bfloat16)
        vg_ref[0] = v_ref[...].astype(jnp.bfloat16)

        barrier_sem = pltpu.get_barrier_semaphore()
        for nbr in (left, right):
            pl.semaphore_signal(
                barrier_sem, inc=1,
                device_id=(nbr,), device_id_type=pl.DeviceIdType.MESH,
            )
        pl.semaphore_wait(barrier_sem, 2)

        for h in range(N_DEV - 1):
            rk = pltpu.make_async_remote_copy(
                src_ref=kg_ref.at[h],
                dst_ref=kg_ref.at[h + 1],
                send_sem=ksend.at[h],
                recv_sem=krecv.at[h],
                device_id=(right,),
                device_id_type=pl.DeviceIdType.MESH,
            )
            rv = pltpu.make_async_remote_copy(
                src_ref=vg_ref.at[h],
                dst_ref=vg_ref.at[h + 1],
                send_sem=vsend.at[h],
                recv_sem=vrecv.at[h],
                device_id=(right,),
                device_id_type=pl.DeviceIdType.MESH,
            )
            rk.start()
            rv.start()
            rk.wait()
            rv.wait()

        wq = wq_ref[...].astype(jnp.bfloat16)
        wo = wo_ref[...].astype(jnp.bfloat16)

        qi = my * SQ + lax.broadcasted_iota(jnp.int32, (SQ, N_DEV * SKV), 0)
        slot = lax.broadcasted_iota(jnp.int32, (SQ, N_DEV * SKV), 1) // SKV
        origin = lax.rem(my - slot + N_DEV, N_DEV)
        ki = origin * SKV + lax.rem(
            lax.broadcasted_iota(jnp.int32, (SQ, N_DEV * SKV), 1), SKV)
        mask = ((jnp.abs(qi - ki) <= WINDOW)
                | (ki < NGLOBAL) | (qi < NGLOBAL))

        kg = kg_ref[...]
        vg = vg_ref[...]

        for b in range(B):
            xb = x_ref[b].astype(jnp.bfloat16)
            q = lax.dot(xb, wq, preferred_element_type=jnp.float32) * SCALE
            ctx_heads = []
            for hh in range(HQ):
                lo, hi = hh * DH, (hh + 1) * DH
                qh = q[:, lo:hi].astype(jnp.bfloat16)
                k_all = jnp.concatenate(
                    [kg[s, b, :, lo:hi] for s in range(N_DEV)], axis=0)
                v_all = jnp.concatenate(
                    [vg[s, b, :, lo:hi] for s in range(N_DEV)], axis=0)
                s = lax.dot_general(
                    qh, k_all, (((1,), (1,)), ((), ())),
                    preferred_element_type=jnp.float32)
                s = jnp.where(mask, s, NEG)
                m = jnp.max(s, axis=1, keepdims=True)
                e = jnp.exp(s - m)
                p = e / jnp.sum(e, axis=1, keepdims=True)
                ctx_heads.append(
                    lax.dot(p.astype(jnp.bfloat16), v_all,
                            preferred_element_type=jnp.float32))
            ctx = jnp.concatenate(ctx_heads, axis=1).astype(jnp.bfloat16)
            out_ref[b, :, :] = lax.dot(
                ctx, wo, preferred_element_type=jnp.float32)

    return pl.pallas_call(
        body,
        out_shape=jax.ShapeDtypeStruct((B, SQ, D_MODEL), jnp.float32),
        in_specs=[pl.BlockSpec(memory_space=pltpu.VMEM)] * 5,
        out_specs=pl.BlockSpec(memory_space=pltpu.VMEM),
        scratch_shapes=[
            pltpu.VMEM((N_DEV, B, SKV, D_QK), jnp.bfloat16),
            pltpu.VMEM((N_DEV, B, SKV, D_QK), jnp.bfloat16),
            pltpu.SemaphoreType.DMA((N_DEV - 1,)),
            pltpu.SemaphoreType.DMA((N_DEV - 1,)),
            pltpu.SemaphoreType.DMA((N_DEV - 1,)),
            pltpu.SemaphoreType.DMA((N_DEV - 1,)),
        ],
        compiler_params=pltpu.CompilerParams(collective_id=0),
    )(x, Wq, k2, v2, Wo)


# device time: 18079 ns/iter; 1.9830x vs baseline; 1.9830x over previous
import functools

import jax
import jax.numpy as jnp
from jax import lax
from jax.experimental import pallas as pl
from jax.experimental.pallas import tpu as pltpu

N_DEV = 4
B = 2
SQ = 256
SKV = 256
HQ = 4
DH = 64
D_MODEL = 512
D_QK = HQ * DH
WINDOW = 128
NG = 32
SCALE = 0.125
NEG = -1e9
BF = jnp.bfloat16
F32 = jnp.float32

KV_SENDS = {
    0: [(1, 0, NG, 0, 0), (1, 128, 256, 1, 1), (3, 0, NG, 2, 0),
        (2, 0, NG, 3, 0)],
    1: [(0, 0, 128, 0, 0), (2, 128, 256, 1, 1)],
    2: [(1, 0, 128, 0, 2), (3, 128, 256, 1, 1)],
    3: [(2, 0, 128, 0, 2)],
}
KV_BLOCKS = {
    0: [([(0, 1, 0, 128)], [(1, 0, 128)], lambda c: 256 + c)],
    1: [([(0, 0, 0, NG), (1, 0, 128, 256)], [(0, 0, NG), (0, 128, 256)],
         lambda c: jnp.where(c < NG, c, c + 96)),
        ([(2, 2, 0, 128)], [(2, 0, 128)], lambda c: 512 + c)],
    2: [([(0, 0, 0, NG), (1, 1, 128, 256), (2, 3, 0, 128)],
         [(1, 128, 256), (3, 0, 128), (0, 0, NG)],
         lambda c: jnp.where(c < 128, 384 + c,
                             jnp.where(c < 256, 640 + c, c - 256)))],
    3: [([(0, 0, 0, NG), (1, 2, 128, 256)], [(2, 128, 256), (0, 0, NG)],
         lambda c: jnp.where(c < 128, 640 + c, c - 128))],
}
PARTIAL = {1: ((-1, 128, 256), 0), 2: ((-1, 0, 256), 1), 3: ((-1, 0, 256), 2)}


def kernel(x, Wq, K_ext, V_ext, Wo):
    k2 = K_ext.reshape(B, SKV, D_QK).astype(BF)
    v2 = V_ext.reshape(B, SKV, D_QK).astype(BF)
    wqb = Wq.astype(BF)
    wob = Wo.astype(BF)

    def body(x_ref, wq_ref, k_ref, v_ref, wo_ref, out_ref,
             kg_ref, vg_ref, qstage_ref, pstage_ref, ppart_ref,
             ksend, krecv, vsend, vrecv, qsend, qrecv, psend, precv):
        my = lax.axis_index("i")

        wq = wq_ref[...]
        wo = wo_ref[...]

        def all_barrier(sem):
            for off in (1, 2, 3):
                pl.semaphore_signal(
                    sem, inc=1,
                    device_id=(lax.rem(my + off, N_DEV),),
                    device_id_type=pl.DeviceIdType.MESH,
                )
            pl.semaphore_wait(sem, N_DEV - 1)

        def recv_wait(dst, sem):
            pltpu.make_async_remote_copy(
                src_ref=dst, dst_ref=dst, send_sem=sem, recv_sem=sem,
                device_id=(0,), device_id_type=pl.DeviceIdType.MESH,
            ).wait_recv()

        def mk_mask(p, pieces, ki_fn):
            n = sum(hi - lo for _, lo, hi in pieces)
            col = lax.broadcasted_iota(jnp.int32, (SQ, n), 1)
            qi = p * SQ + lax.broadcasted_iota(jnp.int32, (SQ, n), 0)
            ki = ki_fn(col)
            keep = (jnp.abs(qi - ki) <= WINDOW) | (ki < NG) | (qi < NG)
            return jnp.where(keep, 0.0, NEG).astype(F32)

        def flash_blk(state, q, pieces, mask):
            for b in range(B):
                kc = jnp.concatenate(
                    [k_ref[b, lo:hi, :] if sl < 0 else kg_ref[sl, b, lo:hi, :]
                     for sl, lo, hi in pieces], axis=0)
                vc = jnp.concatenate(
                    [v_ref[b, lo:hi, :] if sl < 0 else vg_ref[sl, b, lo:hi, :]
                     for sl, lo, hi in pieces], axis=0)
                for hh in range(HQ):
                    d0, d1 = hh * DH, (hh + 1) * DH
                    qh = q[b][:, d0:d1].astype(BF)
                    s = lax.dot_general(
                        qh, kc[:, d0:d1], (((1,), (1,)), ((), ())),
                        preferred_element_type=F32)
                    e = jnp.exp(s + mask)
                    l = jnp.sum(e, axis=1, keepdims=True)
                    acc = lax.dot(e.astype(BF), vc[:, d0:d1],
                                  preferred_element_type=F32)
                    if state[b][hh] is None:
                        state[b][hh] = (l, acc)
                    else:
                        l0, a0 = state[b][hh]
                        state[b][hh] = (l0 + l, a0 + acc)

        def branch(p):
            L = (p - 1) % N_DEV
            started = []

            for in_ref, g_ref, send, recv in (
                    (k_ref, kg_ref, ksend, krecv),
                    (v_ref, vg_ref, vsend, vrecv)):
                for dst, lo, hi, si, ri in KV_SENDS[p]:
                    r = pltpu.make_async_remote_copy(
                        src_ref=in_ref.at[:, lo:hi],
                        dst_ref=g_ref.at[p, :, lo:hi],
                        send_sem=send.at[si], recv_sem=recv.at[ri],
                        device_id=(dst,),
                        device_id_type=pl.DeviceIdType.MESH)
                    r.start()
                    started.append(r)

            if p == 0:
                for b in range(B):
                    q32 = lax.dot(x_ref[b, 0:NG, :].astype(BF), wq,
                                  preferred_element_type=F32) * SCALE
                    qstage_ref[b, :, :] = q32.astype(BF)
                for j, dst in enumerate((1, 2, 3)):
                    r = pltpu.make_async_remote_copy(
                        src_ref=qstage_ref, dst_ref=qstage_ref,
                        send_sem=qsend.at[j], recv_sem=qrecv.at[0],
                        device_id=(dst,),
                        device_id_type=pl.DeviceIdType.MESH)
                    r.start()
                    started.append(r)

            q = [lax.dot(x_ref[b].astype(BF), wq,
                         preferred_element_type=F32) * SCALE
                 for b in range(B)]


            own_pieces = [(-1, 0, SKV)]
            own_mask = mk_mask(p, own_pieces, lambda c: p * SKV + c)
            blk_masks = [mk_mask(p, pieces, ki_fn)
                         for _, pieces, ki_fn in KV_BLOCKS[p]]

            state = [[None] * HQ for _ in range(B)]

            if p != 0:
                recv_wait(qstage_ref, qrecv.at[0])
                (sl, lo, hi), idx = PARTIAL[p]
                for b in range(B):
                    qs = qstage_ref[b]
                    kc = k_ref[b, lo:hi, :]
                    vc = v_ref[b, lo:hi, :]
                    for hh in range(HQ):
                        d0, d1 = hh * DH, (hh + 1) * DH
                        s = lax.dot_general(
                            qs[:, d0:d1], kc[:, d0:d1],
                            (((1,), (1,)), ((), ())),
                            preferred_element_type=F32)
                        e = jnp.exp(s)
                        l = jnp.sum(e, axis=1, keepdims=True)
                        acc = lax.dot(e.astype(BF), vc[:, d0:d1],
                                      preferred_element_type=F32)
                        pstage_ref[b, hh, :, 0:DH] = acc.astype(BF)
                        pstage_ref[b, hh, :, DH:2 * DH] = jnp.broadcast_to(
                            l, (NG, DH)).astype(BF)
                r = pltpu.make_async_remote_copy(
                    src_ref=pstage_ref, dst_ref=ppart_ref.at[idx],
                    send_sem=psend.at[0], recv_sem=precv.at[idx],
                    device_id=(0,), device_id_type=pl.DeviceIdType.MESH)
                r.start()
                started.append(r)

            flash_blk(state, q, own_pieces, own_mask)

            for (recvs, pieces, _), mask in zip(KV_BLOCKS[p], blk_masks):
                for g_ref, recv in ((kg_ref, krecv), (vg_ref, vrecv)):
                    for ri, sl, lo, hi in recvs:
                        recv_wait(g_ref.at[sl, :, lo:hi], recv.at[ri])
                flash_blk(state, q, pieces, mask)

            if p == 0:
                for i in range(3):
                    recv_wait(ppart_ref.at[i], precv.at[i])
            for b in range(B):
                heads = []
                for hh in range(HQ):
                    l, acc = state[b][hh]
                    if p == 0:
                        l_u, a_u = l[0:NG], acc[0:NG]
                        for i in range(3):
                            pp = ppart_ref[i, b, hh]
                            a_u = a_u + pp[:, 0:DH].astype(F32)
                            l_u = l_u + pp[:, DH:DH + 1].astype(F32)
                        ctx_h = jnp.concatenate(
                            [a_u / l_u, acc[NG:] / l[NG:]], axis=0)
                    else:
                        ctx_h = acc / l
                    heads.append(ctx_h)
                ctx = jnp.concatenate(heads, axis=1).astype(BF)
                out_ref[b, :, :] = lax.dot(
                    ctx, wo, preferred_element_type=F32).astype(BF)

            for r in started:
                r.wait_send()

        all_barrier(pltpu.get_barrier_semaphore())

        for p in range(N_DEV):
            @pl.when(my == p)
            def _(p=p):
                branch(p)


    return pl.pallas_call(
        body,
        out_shape=jax.ShapeDtypeStruct((B, SQ, D_MODEL), BF),
        in_specs=[pl.BlockSpec(memory_space=pltpu.VMEM)] * 5,
        out_specs=pl.BlockSpec(memory_space=pltpu.VMEM),
        scratch_shapes=[
            pltpu.VMEM((N_DEV, B, SKV, D_QK), BF),
            pltpu.VMEM((N_DEV, B, SKV, D_QK), BF),
            pltpu.VMEM((B, NG, D_QK), BF),
            pltpu.VMEM((B, HQ, NG, 2 * DH), BF),
            pltpu.VMEM((3, B, HQ, NG, 2 * DH), BF),
            pltpu.SemaphoreType.DMA((4,)),
            pltpu.SemaphoreType.DMA((4,)),
            pltpu.SemaphoreType.DMA((4,)),
            pltpu.SemaphoreType.DMA((4,)),
            pltpu.SemaphoreType.DMA((3,)),
            pltpu.SemaphoreType.DMA((1,)),
            pltpu.SemaphoreType.DMA((1,)),
            pltpu.SemaphoreType.DMA((3,)),
        ],
        compiler_params=pltpu.CompilerParams(collective_id=0),
    )(x, wqb, k2, v2, wob)
